# Initial kernel scaffold; baseline (speedup 1.0000x reference)
import jax
import jax.numpy as jnp
from jax.experimental import pallas as pl


def kernel(x_src, x_dst, edge_index, edge_attr, params):
    raise NotImplementedError("write your pallas kernel here")


# speedup vs baseline: 3.2499x; 3.2499x over previous
import functools

import jax
import jax.numpy as jnp
from jax import lax
from jax.experimental import pallas as pl
from jax.experimental.pallas import tpu as pltpu
from jax.experimental.pallas import tpu_sc as plsc

H = 128
NW = 32
C_IDX = 40
G_PER_GROUP = 5
GROUP = C_IDX * G_PER_GROUP


@functools.partial(jax.jit, static_argnames=())
def _sc_gather(table, idx):
    n, h = table.shape
    e = idx.shape[0]
    per_w = e // NW
    iters = per_w // GROUP
    mesh = plsc.VectorSubcoreMesh(core_axis_name="c", subcore_axis_name="s")

    @functools.partial(
        pl.kernel,
        out_type=jax.ShapeDtypeStruct((e, h), jnp.float32),
        mesh=mesh,
        scratch_types=[
            pltpu.VMEM((per_w,), jnp.int32),
            pltpu.VMEM((GROUP, h), jnp.float32),
            pltpu.SemaphoreType.DMA,
        ],
    )
    def k(table_hbm, idx_hbm, out_hbm, idx_v, rows_v, sem):
        cid = lax.axis_index("c")
        sid = lax.axis_index("s")
        wid = sid * 2 + cid
        base = wid * per_w
        pltpu.sync_copy(idx_hbm.at[pl.ds(base, per_w)], idx_v)

        def body(j, _):
            off = j * GROUP
            cps = []
            for g in range(G_PER_GROUP):
                cp = pltpu.make_async_copy(
                    table_hbm.at[idx_v.at[pl.ds(off + g * C_IDX, C_IDX)]],
                    rows_v.at[pl.ds(g * C_IDX, C_IDX)],
                    sem,
                )
                cp.start()
                cps.append(cp)
            for cp in cps:
                cp.wait()
            pltpu.sync_copy(rows_v, out_hbm.at[pl.ds(base + off, GROUP)])
            return 0

        lax.fori_loop(0, iters, body, 0)

    return k(table, idx)


def _sc_scatter_add(e_new, dst_pad, zeros_n):
    e, h = e_new.shape
    n = zeros_n.shape[0]
    chunks_per_tile = (e // C_IDX) // NW
    pad_chunks = dst_pad.shape[0] // NW
    iters = chunks_per_tile // G_PER_GROUP
    rstep, ra, rb = 640, 400, 240
    mesh = plsc.VectorSubcoreMesh(core_axis_name="c", subcore_axis_name="s")

    @functools.partial(
        pl.kernel,
        out_type=(jax.ShapeDtypeStruct((n, h), jnp.float32),
                  jax.ShapeDtypeStruct((n, h), jnp.float32)),
        mesh=mesh,
        scratch_types=[
            pltpu.VMEM((pad_chunks, C_IDX), jnp.int32),
            pltpu.VMEM((GROUP, h), jnp.float32),
            pltpu.VMEM_SHARED((n, h), jnp.float32),
            pltpu.SemaphoreType.DMA,
        ],
    )
    def k(e_hbm, dst_hbm, z_hbm, out0_hbm, out1_hbm, idx_v, rows_v, agg_sh, sem):
        cid = lax.axis_index("c")
        sid = lax.axis_index("s")
        wid = cid * (NW // 2) + sid
        edge_base = wid * chunks_per_tile * C_IDX
        row0 = sid * rstep
        pltpu.sync_copy(z_hbm.at[pl.ds(row0, ra)], agg_sh.at[pl.ds(row0, ra)])

        @pl.when(sid < 15)
        def _():
            pltpu.sync_copy(z_hbm.at[pl.ds(row0 + ra, rb)],
                            agg_sh.at[pl.ds(row0 + ra, rb)])

        pltpu.sync_copy(dst_hbm.at[pl.ds(wid * pad_chunks, pad_chunks)], idx_v)
        plsc.subcore_barrier()

        def body(j, _):
            off = j * GROUP
            pltpu.sync_copy(e_hbm.at[pl.ds(edge_base + off, GROUP)], rows_v)
            for g in range(G_PER_GROUP):
                pltpu.sync_copy(
                    rows_v.at[pl.ds(g * C_IDX, C_IDX)],
                    agg_sh.at[idx_v.at[j * G_PER_GROUP + g]],
                    add=True,
                )
            return 0

        lax.fori_loop(0, iters, body, 0)
        plsc.subcore_barrier()

        def write_out(out_hbm):
            pltpu.sync_copy(agg_sh.at[pl.ds(row0, ra)],
                            out_hbm.at[pl.ds(row0, ra)])

            @pl.when(sid < 15)
            def _():
                pltpu.sync_copy(agg_sh.at[pl.ds(row0 + ra, rb)],
                                out_hbm.at[pl.ds(row0 + ra, rb)])

        @pl.when(cid == 0)
        def _():
            write_out(out0_hbm)

        @pl.when(cid == 1)
        def _():
            write_out(out1_hbm)

    return k(e_new, dst_pad, zeros_n)


def _tc_mlp(streams, w0_parts, b0, w1, b1, w2, b2, g, beta, res_idx, block):
    n_rows = streams[0].shape[0]
    grid = (n_rows // block,)
    n_s = len(streams)

    def body(*refs):
        in_refs = refs[:n_s]
        w0_refs = refs[n_s:2 * n_s]
        b0r, w1r, b1r, w2r, b2r, gr, br = refs[2 * n_s:2 * n_s + 7]
        out_ref = refs[-1]
        acc = b0r[...]
        for xr, wr in zip(in_refs, w0_refs):
            acc = acc + jnp.dot(xr[...], wr[...],
                                preferred_element_type=jnp.float32)
        hh = acc * jax.nn.sigmoid(acc)
        hh = jnp.dot(hh, w1r[...], preferred_element_type=jnp.float32) + b1r[...]
        hh = hh * jax.nn.sigmoid(hh)
        hh = jnp.dot(hh, w2r[...], preferred_element_type=jnp.float32) + b2r[...]
        mu = jnp.mean(hh, axis=-1, keepdims=True)
        var = jnp.mean((hh - mu) ** 2, axis=-1, keepdims=True)
        o = (hh - mu) / jnp.sqrt(var + 1e-5) * gr[...] + br[...]
        if res_idx is not None:
            o = o + in_refs[res_idx][...]
        out_ref[...] = o

    in_specs = [pl.BlockSpec((block, s.shape[1]), lambda i: (i, 0))
                for s in streams]
    w_specs = [pl.BlockSpec(w.shape, lambda i: (0, 0))
               for w in (*w0_parts, b0, w1, b1, w2, b2, g, beta)]
    return pl.pallas_call(
        body,
        grid=grid,
        in_specs=in_specs + w_specs,
        out_specs=pl.BlockSpec((block, H), lambda i: (i, 0)),
        out_shape=jax.ShapeDtypeStruct((n_rows, H), jnp.float32),
    )(*streams, *w0_parts, b0, w1, b1, w2, b2, g, beta)


def _split_mlp_params(p, n_streams):
    w0, b0, w1, b1, w2, b2, g, beta = p
    din = w0.shape[0]
    step = din // n_streams
    parts = tuple(w0[i * step:(i + 1) * step] for i in range(n_streams))
    r = lambda v: v.reshape(1, -1)
    return parts, r(b0), w1, r(b1), w2, r(b2), r(g), r(beta)


def kernel(x_src, x_dst, edge_index, edge_attr, params):
    e = edge_attr.shape[0]
    n = x_dst.shape[0]
    src = edge_index[0]
    dst = edge_index[1]
    chunks_per_tile = (e // C_IDX) // NW
    dst_pad = jnp.pad(
        dst.reshape(NW, chunks_per_tile, C_IDX),
        ((0, 0), (0, 128 - chunks_per_tile), (0, 0)),
    ).reshape(NW * 128, C_IDX)
    zeros_n = jnp.zeros((n, H), jnp.float32)

    p0, b0, w1, b1, w2, b2, g, beta = _split_mlp_params(params["edge_enc"], 1)
    ea = _tc_mlp([edge_attr], p0, b0, w1, b1, w2, b2, g, beta,
                 res_idx=None, block=2000)

    xj = _sc_gather(x_src, src)

    xd = x_dst
    for blk in params["blocks"]:
        xi = _sc_gather(xd, dst)
        ep, eb0, ew1, eb1, ew2, eb2, eg, ebeta = _split_mlp_params(
            blk["edge_mlp"], 3)
        e_new = _tc_mlp([xi, xj, ea], ep, eb0, ew1, eb1, ew2, eb2, eg, ebeta,
                        res_idx=2, block=2000)
        agg0, agg1 = _sc_scatter_add(e_new, dst_pad, zeros_n)
        np_, nb0, nw1, nb1, nw2, nb2, ng, nbeta = _split_mlp_params(
            blk["node_mlp"], 2)
        xd = _tc_mlp([xd, agg0, agg1], (np_[0], np_[1], np_[1]),
                     nb0, nw1, nb1, nw2, nb2, ng, nbeta,
                     res_idx=0, block=2000)
        ea = e_new
    return xd


# speedup vs baseline: 3.3989x; 1.0458x over previous
import functools

import jax
import jax.numpy as jnp
from jax import lax
from jax.experimental import pallas as pl
from jax.experimental.pallas import tpu as pltpu
from jax.experimental.pallas import tpu_sc as plsc

H = 128
NW = 32
C_IDX = 40
G_PER_GROUP = 5
GROUP = C_IDX * G_PER_GROUP
RING = 5


@functools.partial(jax.jit, static_argnames=())
def _sc_gather(table, idx):
    n, h = table.shape
    e = idx.shape[0]
    per_w = e // NW
    iters = per_w // GROUP
    mesh = plsc.VectorSubcoreMesh(core_axis_name="c", subcore_axis_name="s")

    assert iters % 2 == 1 and iters >= 3

    @functools.partial(
        pl.kernel,
        out_type=jax.ShapeDtypeStruct((e, h), jnp.float32),
        mesh=mesh,
        scratch_types=[
            pltpu.VMEM((per_w,), jnp.int32),
            pltpu.VMEM((GROUP, h), jnp.float32),
            pltpu.VMEM((GROUP, h), jnp.float32),
            pltpu.SemaphoreType.DMA,
            pltpu.SemaphoreType.DMA,
            pltpu.SemaphoreType.DMA,
            pltpu.SemaphoreType.DMA,
        ],
    )
    def k(table_hbm, idx_hbm, out_hbm, idx_v, b0, b1, sg0, sg1, sw0, sw1):
        cid = lax.axis_index("c")
        sid = lax.axis_index("s")
        wid = sid * 2 + cid
        base = wid * per_w
        pltpu.sync_copy(idx_hbm.at[pl.ds(base, per_w)], idx_v)

        def fire_g(a, buf, sem):
            off = a * GROUP
            for g in range(G_PER_GROUP):
                pltpu.async_copy(
                    table_hbm.at[idx_v.at[pl.ds(off + g * C_IDX, C_IDX)]],
                    buf.at[pl.ds(g * C_IDX, C_IDX)], sem)

        def drain_g(buf, sem):
            pltpu.make_async_copy(table_hbm.at[pl.ds(0, GROUP)], buf, sem).wait()

        def fire_w(a, buf, sem):
            pltpu.async_copy(buf, out_hbm.at[pl.ds(base + a * GROUP, GROUP)], sem)

        def drain_w(buf, sem):
            pltpu.make_async_copy(buf, out_hbm.at[pl.ds(base, GROUP)], sem).wait()

        fire_g(0, b0, sg0)

        def body(j2, _):
            a = 2 * j2

            @pl.when(j2 > 0)
            def _():
                drain_w(b1, sw1)

            fire_g(a + 1, b1, sg1)
            drain_g(b0, sg0)
            fire_w(a, b0, sw0)
            drain_w(b0, sw0)
            fire_g(a + 2, b0, sg0)
            drain_g(b1, sg1)
            fire_w(a + 1, b1, sw1)
            return 0

        lax.fori_loop(0, (iters - 1) // 2, body, 0)
        drain_g(b0, sg0)
        fire_w(iters - 1, b0, sw0)
        drain_w(b1, sw1)
        drain_w(b0, sw0)

    return k(table, idx)


def _sc_scatter_add(e_new, dst_pad, zeros_n):
    e, h = e_new.shape
    n = zeros_n.shape[0]
    chunks_per_tile = (e // C_IDX) // NW
    pad_chunks = dst_pad.shape[0] // NW
    assert chunks_per_tile % RING == 0
    rstep, ra, rb = 640, 400, 240
    mesh = plsc.VectorSubcoreMesh(core_axis_name="c", subcore_axis_name="s")

    @functools.partial(
        pl.kernel,
        out_type=(jax.ShapeDtypeStruct((n, h), jnp.float32),
                  jax.ShapeDtypeStruct((n, h), jnp.float32)),
        mesh=mesh,
        scratch_types=[
            pltpu.VMEM((pad_chunks, C_IDX), jnp.int32),
            pltpu.VMEM((RING * C_IDX, h), jnp.float32),
            pltpu.VMEM_SHARED((n, h), jnp.float32),
            [pltpu.SemaphoreType.DMA] * RING,
            [pltpu.SemaphoreType.DMA] * RING,
        ],
    )
    def k(e_hbm, dst_hbm, z_hbm, out0_hbm, out1_hbm,
          idx_v, rows_v, agg_sh, sl, ss):
        cid = lax.axis_index("c")
        sid = lax.axis_index("s")
        wid = cid * (NW // 2) + sid
        edge_base = wid * chunks_per_tile * C_IDX
        row0 = sid * rstep
        pltpu.sync_copy(z_hbm.at[pl.ds(row0, ra)], agg_sh.at[pl.ds(row0, ra)])

        @pl.when(sid < 15)
        def _():
            pltpu.sync_copy(z_hbm.at[pl.ds(row0 + ra, rb)],
                            agg_sh.at[pl.ds(row0 + ra, rb)])

        pltpu.sync_copy(dst_hbm.at[pl.ds(wid * pad_chunks, pad_chunks)], idx_v)
        plsc.subcore_barrier()

        slot = lambda si: rows_v.at[pl.ds(si * C_IDX, C_IDX)]

        def fire_l(c, si):
            pltpu.async_copy(e_hbm.at[pl.ds(edge_base + c * C_IDX, C_IDX)],
                             slot(si), sl[si])

        def drain_l(si):
            pltpu.make_async_copy(e_hbm.at[pl.ds(0, C_IDX)], slot(si),
                                  sl[si]).wait()

        def fire_s(c, si):
            pltpu.async_copy(slot(si), agg_sh.at[idx_v.at[c]], ss[si],
                             add=True)

        def drain_s(si):
            pltpu.make_async_copy(e_hbm.at[pl.ds(0, C_IDX)], slot(si),
                                  ss[si]).wait()

        def body(j, _):
            for si in range(RING):
                c = RING * j + si

                @pl.when(j > 0)
                def _():
                    drain_s(si)

                fire_l(c, si)
                sp = (si - 1) % RING

                @pl.when(c > 0)
                def _():
                    drain_l(sp)
                    fire_s(c - 1, sp)

            return 0

        lax.fori_loop(0, chunks_per_tile // RING, body, 0)
        drain_l(RING - 1)
        fire_s(chunks_per_tile - 1, RING - 1)
        for si in range(RING):
            drain_s(si)
        plsc.subcore_barrier()

        def write_out(out_hbm):
            pltpu.sync_copy(agg_sh.at[pl.ds(row0, ra)],
                            out_hbm.at[pl.ds(row0, ra)])

            @pl.when(sid < 15)
            def _():
                pltpu.sync_copy(agg_sh.at[pl.ds(row0 + ra, rb)],
                                out_hbm.at[pl.ds(row0 + ra, rb)])

        @pl.when(cid == 0)
        def _():
            write_out(out0_hbm)

        @pl.when(cid == 1)
        def _():
            write_out(out1_hbm)

    return k(e_new, dst_pad, zeros_n)


def _tc_mlp(streams, w0_parts, b0, w1, b1, w2, b2, g, beta, res_idx, block):
    n_rows = streams[0].shape[0]
    grid = (n_rows // block,)
    n_s = len(streams)

    def body(*refs):
        in_refs = refs[:n_s]
        w0_refs = refs[n_s:2 * n_s]
        b0r, w1r, b1r, w2r, b2r, gr, br = refs[2 * n_s:2 * n_s + 7]
        out_ref = refs[-1]
        acc = b0r[...]
        for xr, wr in zip(in_refs, w0_refs):
            acc = acc + jnp.dot(xr[...], wr[...],
                                preferred_element_type=jnp.float32)
        hh = acc * jax.nn.sigmoid(acc)
        hh = jnp.dot(hh, w1r[...], preferred_element_type=jnp.float32) + b1r[...]
        hh = hh * jax.nn.sigmoid(hh)
        hh = jnp.dot(hh, w2r[...], preferred_element_type=jnp.float32) + b2r[...]
        mu = jnp.mean(hh, axis=-1, keepdims=True)
        var = jnp.mean((hh - mu) ** 2, axis=-1, keepdims=True)
        o = (hh - mu) / jnp.sqrt(var + 1e-5) * gr[...] + br[...]
        if res_idx is not None:
            o = o + in_refs[res_idx][...]
        out_ref[...] = o

    in_specs = [pl.BlockSpec((block, s.shape[1]), lambda i: (i, 0))
                for s in streams]
    w_specs = [pl.BlockSpec(w.shape, lambda i: (0, 0))
               for w in (*w0_parts, b0, w1, b1, w2, b2, g, beta)]
    return pl.pallas_call(
        body,
        grid=grid,
        in_specs=in_specs + w_specs,
        out_specs=pl.BlockSpec((block, H), lambda i: (i, 0)),
        out_shape=jax.ShapeDtypeStruct((n_rows, H), jnp.float32),
    )(*streams, *w0_parts, b0, w1, b1, w2, b2, g, beta)


def _split_mlp_params(p, n_streams):
    w0, b0, w1, b1, w2, b2, g, beta = p
    din = w0.shape[0]
    step = din // n_streams
    parts = tuple(w0[i * step:(i + 1) * step] for i in range(n_streams))
    r = lambda v: v.reshape(1, -1)
    return parts, r(b0), w1, r(b1), w2, r(b2), r(g), r(beta)


def kernel(x_src, x_dst, edge_index, edge_attr, params):
    e = edge_attr.shape[0]
    n = x_dst.shape[0]
    src = edge_index[0]
    dst = edge_index[1]
    chunks_per_tile = (e // C_IDX) // NW
    dst_pad = jnp.pad(
        dst.reshape(NW, chunks_per_tile, C_IDX),
        ((0, 0), (0, 128 - chunks_per_tile), (0, 0)),
    ).reshape(NW * 128, C_IDX)
    zeros_n = jnp.zeros((n, H), jnp.float32)

    p0, b0, w1, b1, w2, b2, g, beta = _split_mlp_params(params["edge_enc"], 1)
    ea = _tc_mlp([edge_attr], p0, b0, w1, b1, w2, b2, g, beta,
                 res_idx=None, block=2000)

    xj = _sc_gather(x_src, src)

    xd = x_dst
    for blk in params["blocks"]:
        xi = _sc_gather(xd, dst)
        ep, eb0, ew1, eb1, ew2, eb2, eg, ebeta = _split_mlp_params(
            blk["edge_mlp"], 3)
        e_new = _tc_mlp([xi, xj, ea], ep, eb0, ew1, eb1, ew2, eb2, eg, ebeta,
                        res_idx=2, block=2000)
        agg0, agg1 = _sc_scatter_add(e_new, dst_pad, zeros_n)
        np_, nb0, nw1, nb1, nw2, nb2, ng, nbeta = _split_mlp_params(
            blk["node_mlp"], 2)
        xd = _tc_mlp([xd, agg0, agg1], (np_[0], np_[1], np_[1]),
                     nb0, nw1, nb1, nw2, nb2, ng, nbeta,
                     res_idx=0, block=2000)
        ea = e_new
    return xd
